# baseline (speedup 1.0000x reference)
import functools

import jax
import jax.numpy as jnp
from jax import lax
from jax.experimental import pallas as pl
from jax.experimental.pallas import tpu as pltpu
from jax.experimental.pallas import tpu_sc as plsc

D_M = 1024
N_POS = 4096
N_BATCH = 4
N_SRC = 4
_SPLIT = 3072
_SPAN = N_POS - _SPLIT
_INFO = plsc.get_sparse_core_info()
_NC, _NS, _L = _INFO.num_cores, _INFO.num_subcores, _INFO.num_lanes
_NW = _NC * _NS
_P_PER_W = _SPAN // _NW
_P_CHUNK = 4
_N_CHUNKS = _P_PER_W // _P_CHUNK
_NVEC = D_M // _L
_NBUF = 5
_ID = 3
_OD = 2
_BP = 512


def _sc_body(x_hbm, emb_hbm, ids_rep_hbm, out_hbm, *refs):
    xbufs = refs[:_NBUF]
    idx_v = refs[_NBUF]
    emb_v = refs[_NBUF + 1]
    isems = refs[_NBUF + 2:2 * _NBUF + 2]
    osems = refs[2 * _NBUF + 2:3 * _NBUF + 2]

    wid = lax.axis_index("s") * _NC + lax.axis_index("c")
    base_p = wid * _P_PER_W

    def in_copy(c):
        b = c % _NBUF
        return pltpu.make_async_copy(
            x_hbm.at[pl.ds(_SPLIT + base_p + c * _P_CHUNK, _P_CHUNK)],
            xbufs[b], isems[b])

    def out_copy(c):
        b = c % _NBUF
        return pltpu.make_async_copy(
            xbufs[b], out_hbm.at[pl.ds(base_p + c * _P_CHUNK, _P_CHUNK)],
            osems[b])

    for c in range(_ID):
        in_copy(c).start()
    pltpu.sync_copy(ids_rep_hbm.at[pl.ds(_SPLIT + base_p, _P_PER_W)], idx_v)
    pltpu.sync_copy(emb_hbm, emb_v)
    for c in range(_N_CHUNKS):
        b = c % _NBUF
        in_copy(c).wait()
        xbuf = xbufs[b]

        idvecs = [idx_v[c * _P_CHUNK + p, :] for p in range(_P_CHUNK)]

        def col(j, carry, xbuf=xbuf, idvecs=idvecs):
            off = pl.ds(j * _L, _L)
            evs = [emb_v[s, off] for s in range(N_SRC)]
            for p in range(_P_CHUNK):
                iv = idvecs[p]
                ev = evs[N_SRC - 1]
                for s in range(N_SRC - 2, -1, -1):
                    ev = jnp.where(iv == s, evs[s], ev)
                for bb in range(N_BATCH):
                    xbuf[p, bb, off] = xbuf[p, bb, off] + ev
            return carry

        lax.fori_loop(0, _NVEC, col, 0)
        out_copy(c).start()
        if c >= _OD:
            out_copy(c - _OD).wait()
        if c + _ID < _N_CHUNKS:
            in_copy(c + _ID).start()
    for c in range(_N_CHUNKS - _OD, _N_CHUNKS):
        out_copy(c).wait()


def _tc_body(ids_ref, emb_ref, x_ref, o_ref):
    ids = ids_ref[...]
    emb = emb_ref[...]
    n_sources, d = emb.shape
    enc = jnp.zeros((ids.shape[0], d), jnp.float32)
    for s in range(n_sources):
        enc = jnp.where(ids == s, emb[s].reshape(1, d), enc)
    o_ref[...] = x_ref[...] + enc[:, None, :]


@functools.partial(jax.jit, static_argnums=())
def _hybrid_call(x, emb, src_ids):
    mesh = plsc.VectorSubcoreMesh(core_axis_name="c", subcore_axis_name="s")
    scratch = [pltpu.VMEM((_P_CHUNK, N_BATCH, D_M), jnp.float32)
               for _ in range(_NBUF)]
    scratch += [
        pltpu.VMEM((_P_PER_W, _L), jnp.int32),
        pltpu.VMEM((N_SRC, D_M), jnp.float32),
    ]
    scratch += [pltpu.SemaphoreType.DMA for _ in range(2 * _NBUF)]
    sc = pl.kernel(
        _sc_body,
        mesh=mesh,
        out_type=jax.ShapeDtypeStruct((_SPAN, N_BATCH, D_M), jnp.float32),
        scratch_types=scratch,
    )
    ids_rep = jnp.broadcast_to(src_ids[:, None], (N_POS, _L))
    sc_out = sc(x, emb, ids_rep)

    ids2 = src_ids.reshape(N_POS, 1)
    tc_out = pl.pallas_call(
        _tc_body,
        grid=(_SPLIT // _BP,),
        in_specs=[
            pl.BlockSpec((_BP, 1), lambda i: (i, 0)),
            pl.BlockSpec(emb.shape, lambda i: (0, 0)),
            pl.BlockSpec((_BP, N_BATCH, D_M), lambda i: (i, 0, 0)),
        ],
        out_specs=pl.BlockSpec((_BP, N_BATCH, D_M), lambda i: (i, 0, 0)),
        out_shape=jax.ShapeDtypeStruct(x.shape, x.dtype),
    )(ids2, emb, x)
    return lax.dynamic_update_slice(tc_out, sc_out, (_SPLIT, 0, 0))


def kernel(x, emb, src_ids):
    return _hybrid_call(x, emb, src_ids)


# speedup vs baseline: 1.1173x; 1.1173x over previous
import functools

import jax
import jax.numpy as jnp
from jax import lax
from jax.experimental import pallas as pl
from jax.experimental.pallas import tpu as pltpu
from jax.experimental.pallas import tpu_sc as plsc

D_M = 1024
N_POS = 4096
N_BATCH = 4
N_SRC = 4
_INFO = plsc.get_sparse_core_info()
_NC, _NS, _L = _INFO.num_cores, _INFO.num_subcores, _INFO.num_lanes
_NW = _NC * _NS
_P_PER_W = N_POS // _NW
_P_CHUNK = 8
_N_CHUNKS = _P_PER_W // _P_CHUNK
_NVEC = D_M // _L
_NBUF = 3
_ID = 2
_OD = 1


def _sc_body(x_hbm, emb_hbm, ids_rep_hbm, out_hbm, *refs):
    xbufs = refs[:_NBUF]
    idx_v = refs[_NBUF]
    emb_v = refs[_NBUF + 1]
    isems = refs[_NBUF + 2:2 * _NBUF + 2]
    osems = refs[2 * _NBUF + 2:3 * _NBUF + 2]

    wid = lax.axis_index("s") * _NC + lax.axis_index("c")
    base_p = wid * _P_PER_W

    def in_copy(c):
        b = c % _NBUF
        return pltpu.make_async_copy(
            x_hbm.at[pl.ds(base_p + c * _P_CHUNK, _P_CHUNK)], xbufs[b], isems[b])

    def out_copy(c):
        b = c % _NBUF
        return pltpu.make_async_copy(
            xbufs[b], out_hbm.at[pl.ds(base_p + c * _P_CHUNK, _P_CHUNK)],
            osems[b])

    for c in range(_ID):
        in_copy(c).start()
    pltpu.sync_copy(ids_rep_hbm.at[pl.ds(base_p, _P_PER_W)], idx_v)
    pltpu.sync_copy(emb_hbm, emb_v)
    for c in range(_N_CHUNKS):
        b = c % _NBUF
        in_copy(c).wait()
        xbuf = xbufs[b]

        idvecs = [idx_v[c * _P_CHUNK + p, :] for p in range(_P_CHUNK)]

        def col(j, carry, xbuf=xbuf, idvecs=idvecs):
            off = pl.ds(j * _L, _L)
            evs = [emb_v[s, off] for s in range(N_SRC)]
            for p in range(_P_CHUNK):
                iv = idvecs[p]
                ev = evs[N_SRC - 1]
                for s in range(N_SRC - 2, -1, -1):
                    ev = jnp.where(iv == s, evs[s], ev)
                for bb in range(N_BATCH):
                    xbuf[p, bb, off] = xbuf[p, bb, off] + ev
            return carry

        lax.fori_loop(0, _NVEC, col, 0)
        out_copy(c).start()
        if c >= _OD:
            out_copy(c - _OD).wait()
        if c + _ID < _N_CHUNKS:
            in_copy(c + _ID).start()
    for c in range(_N_CHUNKS - _OD, _N_CHUNKS):
        out_copy(c).wait()


@functools.partial(jax.jit, static_argnums=())
def _sc_call(x, emb, src_ids):
    mesh = plsc.VectorSubcoreMesh(core_axis_name="c", subcore_axis_name="s")
    scratch = [pltpu.VMEM((_P_CHUNK, N_BATCH, D_M), jnp.float32)
               for _ in range(_NBUF)]
    scratch += [
        pltpu.VMEM((_P_PER_W, _L), jnp.int32),
        pltpu.VMEM((N_SRC, D_M), jnp.float32),
    ]
    scratch += [pltpu.SemaphoreType.DMA for _ in range(2 * _NBUF)]
    f = pl.kernel(
        _sc_body,
        mesh=mesh,
        out_type=jax.ShapeDtypeStruct((N_POS, N_BATCH, D_M), jnp.float32),
        scratch_types=scratch,
    )
    ids_rep = jnp.broadcast_to(src_ids[:, None], (N_POS, _L))
    return f(x, emb, ids_rep)


def kernel(x, emb, src_ids):
    return _sc_call(x, emb, src_ids)


# speedup vs baseline: 1.1425x; 1.0225x over previous
import functools

import jax
import jax.numpy as jnp
from jax import lax
from jax.experimental import pallas as pl
from jax.experimental.pallas import tpu as pltpu
from jax.experimental.pallas import tpu_sc as plsc

D_M = 1024
N_POS = 4096
N_BATCH = 4
N_SRC = 4
_INFO = plsc.get_sparse_core_info()
_NC, _NS, _L = _INFO.num_cores, _INFO.num_subcores, _INFO.num_lanes
_NW = _NC * _NS
_P_PER_W = N_POS // _NW
_P_CHUNK = 4
_N_CHUNKS = _P_PER_W // _P_CHUNK
_NVEC = D_M // _L
_NBUF = 6
_ID = 3
_OD = 3


def _sc_body(x_hbm, emb_hbm, ids_rep_hbm, out_hbm, *refs):
    xbufs = refs[:_NBUF]
    idx_v = refs[_NBUF]
    emb_v = refs[_NBUF + 1]
    isems = refs[_NBUF + 2:2 * _NBUF + 2]
    osems = refs[2 * _NBUF + 2:3 * _NBUF + 2]

    wid = lax.axis_index("s") * _NC + lax.axis_index("c")
    base_p = wid * _P_PER_W

    def in_copy(c):
        b = c % _NBUF
        return pltpu.make_async_copy(
            x_hbm.at[pl.ds(base_p + c * _P_CHUNK, _P_CHUNK)], xbufs[b], isems[b])

    def out_copy(c):
        b = c % _NBUF
        return pltpu.make_async_copy(
            xbufs[b], out_hbm.at[pl.ds(base_p + c * _P_CHUNK, _P_CHUNK)],
            osems[b])

    for c in range(_ID):
        in_copy(c).start()
    pltpu.sync_copy(ids_rep_hbm.at[pl.ds(base_p, _P_PER_W)], idx_v)
    pltpu.sync_copy(emb_hbm, emb_v)
    for c in range(_N_CHUNKS):
        b = c % _NBUF
        in_copy(c).wait()
        xbuf = xbufs[b]

        idvecs = [idx_v[c * _P_CHUNK + p, :] for p in range(_P_CHUNK)]

        def col(j, carry, xbuf=xbuf, idvecs=idvecs):
            off = pl.ds(j * _L, _L)
            evs = [emb_v[s, off] for s in range(N_SRC)]
            for p in range(_P_CHUNK):
                iv = idvecs[p]
                ev = evs[N_SRC - 1]
                for s in range(N_SRC - 2, -1, -1):
                    ev = jnp.where(iv == s, evs[s], ev)
                for bb in range(N_BATCH):
                    xbuf[p, bb, off] = xbuf[p, bb, off] + ev
            return carry

        lax.fori_loop(0, _NVEC, col, 0)
        out_copy(c).start()
        if c >= _OD:
            out_copy(c - _OD).wait()
        if c + _ID < _N_CHUNKS:
            in_copy(c + _ID).start()
    for c in range(_N_CHUNKS - _OD, _N_CHUNKS):
        out_copy(c).wait()


@functools.partial(jax.jit, static_argnums=())
def _sc_call(x, emb, src_ids):
    mesh = plsc.VectorSubcoreMesh(core_axis_name="c", subcore_axis_name="s")
    scratch = [pltpu.VMEM((_P_CHUNK, N_BATCH, D_M), jnp.float32)
               for _ in range(_NBUF)]
    scratch += [
        pltpu.VMEM((_P_PER_W, _L), jnp.int32),
        pltpu.VMEM((N_SRC, D_M), jnp.float32),
    ]
    scratch += [pltpu.SemaphoreType.DMA for _ in range(2 * _NBUF)]
    f = pl.kernel(
        _sc_body,
        mesh=mesh,
        out_type=jax.ShapeDtypeStruct((N_POS, N_BATCH, D_M), jnp.float32),
        scratch_types=scratch,
    )
    ids_rep = jnp.broadcast_to(src_ids[:, None], (N_POS, _L))
    return f(x, emb, ids_rep)


def kernel(x, emb, src_ids):
    return _sc_call(x, emb, src_ids)
